# baseline (speedup 1.0000x reference)
import functools

import jax
import jax.numpy as jnp
from jax import lax
from jax.experimental import pallas as pl
from jax.experimental.pallas import tpu as pltpu
from jax.experimental.pallas import tpu_sc as plsc

_B = 16384
_D = 64
_TOT = 2 * _B
_CH = 128
_NW = 32
_RPW = _TOT // _NW
_CPW = _RPW // _CH
_L = 16


def _sc_sumsq(ent, rel, h2, r2, t2):
    mesh = plsc.VectorSubcoreMesh(core_axis_name="c", subcore_axis_name="s")

    @functools.partial(
        pl.kernel,
        mesh=mesh,
        out_type=jax.ShapeDtypeStruct((_TOT,), jnp.float32),
        compiler_params=pltpu.CompilerParams(
            use_tc_tiling_on_sc=False, needs_layout_passes=False),
        scratch_types=[
            pltpu.VMEM((_CPW, _CH), jnp.int32),
            pltpu.VMEM((_CPW, _CH), jnp.int32),
            pltpu.VMEM((_CPW, _CH), jnp.int32),
            pltpu.VMEM((_CH, _D), jnp.float32),
            pltpu.VMEM((_CH, _D), jnp.float32),
            pltpu.VMEM((_CH, _D), jnp.float32),
            pltpu.VMEM((_RPW,), jnp.float32),
            pltpu.SemaphoreType.DMA,
        ],
    )
    def k(ent_hbm, rel_hbm, h_hbm, r_hbm, t_hbm, out_hbm,
          hidx, ridx, tidx, hbuf, rbuf, tbuf, ss, sem):
        wid = lax.axis_index("s") * 2 + lax.axis_index("c")
        ibase = wid * _CPW
        pltpu.sync_copy(h_hbm.at[pl.ds(ibase, _CPW)], hidx)
        pltpu.sync_copy(r_hbm.at[pl.ds(ibase, _CPW)], ridx)
        pltpu.sync_copy(t_hbm.at[pl.ds(ibase, _CPW)], tidx)
        lanes = lax.iota(jnp.int32, _L)

        def chunk(c, carry):
            cp1 = pltpu.async_copy(ent_hbm.at[hidx.at[c]], hbuf, sem)
            cp2 = pltpu.async_copy(rel_hbm.at[ridx.at[c]], rbuf, sem)
            cp3 = pltpu.async_copy(ent_hbm.at[tidx.at[c]], tbuf, sem)
            cp1.wait()
            cp2.wait()
            cp3.wait()
            for g in range(_CH // _L):
                rows = lanes + (g * _L)
                acc = jnp.zeros((_L,), jnp.float32)
                for dd in range(_D):
                    col = jnp.full((_L,), dd, jnp.int32)
                    hv = plsc.load_gather(hbuf, [rows, col])
                    rv = plsc.load_gather(rbuf, [rows, col])
                    tv = plsc.load_gather(tbuf, [rows, col])
                    e = hv + rv - tv
                    acc = acc + e * e
                ss[pl.ds(c * _CH + g * _L, _L)] = acc
            return carry

        lax.fori_loop(0, _CPW, chunk, 0)
        pltpu.sync_copy(ss, out_hbm.at[pl.ds(wid * _RPW, _RPW)])

    return k(ent, rel, h2, r2, t2)


def _tc_finish(ss2, pp2, ap2, params):
    def body(par_ref, ss_ref, pp_ref, ap_ref, o_ref):
        alpha = par_ref[0]
        beta = par_ref[1]
        l1 = par_ref[2]
        l2 = par_ref[3]
        l3 = par_ref[4]
        pos = jnp.sqrt(ss_ref[0] + 1e-12)
        neg = jnp.sqrt(ss_ref[1] + 1e-12)
        d = pos - neg + 1.0
        lt = jnp.where(d < 0, 1.0 + beta, alpha)
        cw = l1 * lt + l2 * pp_ref[...] + l3 * (1.0 / (1.0 + jnp.exp(-ap_ref[...])))
        o_ref[0, 0] = jnp.sum(jnp.maximum(d * cw, 0.0)) * (1.0 / _B)

    out = pl.pallas_call(
        body,
        out_shape=jax.ShapeDtypeStruct((1, 1), jnp.float32),
        in_specs=[
            pl.BlockSpec(memory_space=pltpu.SMEM),
            pl.BlockSpec(memory_space=pltpu.VMEM),
            pl.BlockSpec(memory_space=pltpu.VMEM),
            pl.BlockSpec(memory_space=pltpu.VMEM),
        ],
        out_specs=pl.BlockSpec(memory_space=pltpu.SMEM),
    )(params, ss2, pp2, ap2)
    return out[0, 0]


def kernel(posX, negX, entityEmbedding, relationEmbedding, PP, AP,
           alpha, beta, sigma, lambda1, lambda2, lambda3):
    h2 = jnp.concatenate([posX[:, 0], negX[:, 0]]).reshape(_TOT // _CH, _CH)
    r2 = jnp.concatenate([posX[:, 1], negX[:, 1]]).reshape(_TOT // _CH, _CH)
    t2 = jnp.concatenate([posX[:, 2], negX[:, 2]]).reshape(_TOT // _CH, _CH)
    ss = _sc_sumsq(entityEmbedding, relationEmbedding, h2, r2, t2)
    ss2 = ss.reshape(2, 128, 128)
    pp2 = PP.reshape(128, 128)
    ap2 = AP.reshape(128, 128)
    params = jnp.stack([alpha, beta, lambda1, lambda2, lambda3]).astype(jnp.float32)
    return _tc_finish(ss2, pp2, ap2, params)


# speedup vs baseline: 1.0034x; 1.0034x over previous
import functools

import jax
import jax.numpy as jnp
from jax import lax
from jax.experimental import pallas as pl
from jax.experimental.pallas import tpu as pltpu
from jax.experimental.pallas import tpu_sc as plsc

_B = 16384
_D = 64
_TOT = 2 * _B
_CH = 128
_NW = 32
_RPW = _TOT // _NW
_HPW = _RPW // 2
_CPW = _RPW // _CH
_L = 16


def _vsqrt(x):
    i = plsc.bitcast(x, jnp.int32)
    i = jnp.int32(0x5F3759DF) - (i >> 1)
    y = plsc.bitcast(i, jnp.float32)
    for _ in range(3):
        y = y * (1.5 - 0.5 * x * y * y)
    return x * y


def _splat(ref, i):
    return plsc.load_gather(ref, [jnp.full((_L,), i, jnp.int32)])


def _sc_loss(entv, relv, hh2, rh2, th2, hp1, rp1, tp1, pp, ap, prm):
    mesh = plsc.VectorSubcoreMesh(core_axis_name="c", subcore_axis_name="s")

    @functools.partial(
        pl.kernel,
        mesh=mesh,
        out_type=(jax.ShapeDtypeStruct((_NW * _L,), jnp.float32),
                  jax.ShapeDtypeStruct((_TOT,), jnp.float32)),
        compiler_params=pltpu.CompilerParams(needs_layout_passes=False),
        scratch_types=[
            pltpu.VMEM((_CPW, _CH), jnp.int32),
            pltpu.VMEM((_CPW, _CH), jnp.int32),
            pltpu.VMEM((_CPW, _CH), jnp.int32),
            pltpu.VMEM((_RPW,), jnp.int32),
            pltpu.VMEM((_RPW,), jnp.int32),
            pltpu.VMEM((_RPW,), jnp.int32),
            pltpu.VMEM((_CH, 2 * _D), jnp.float32),
            pltpu.VMEM((_CH, 2 * _D), jnp.float32),
            pltpu.VMEM((_CH, 2 * _D), jnp.float32),
            pltpu.VMEM((_CH, 2 * _D), jnp.float32),
            pltpu.VMEM((_CH, 2 * _D), jnp.float32),
            pltpu.VMEM((_CH, 2 * _D), jnp.float32),
            pltpu.VMEM((_RPW,), jnp.float32),
            pltpu.VMEM((_HPW,), jnp.float32),
            pltpu.VMEM((_HPW,), jnp.float32),
            pltpu.VMEM((_L,), jnp.float32),
            pltpu.VMEM((_L,), jnp.float32),
            pltpu.SemaphoreType.DMA,
            pltpu.SemaphoreType.DMA,
        ],
    )
    def k(ent_hbm, rel_hbm, hh_hbm, rh_hbm, th_hbm, hp_hbm, rp_hbm, tp_hbm,
          pp_hbm, ap_hbm, prm_hbm, out_hbm,
          hh, rh, th, hp, rp, tp, ha, ra, ta, hb, rb, tb,
          ss, ppv, apv, prm_v, accv, semA, semB):
        wid = lax.axis_index("s") * 2 + lax.axis_index("c")
        ib = wid * _CPW
        pltpu.sync_copy(hh_hbm.at[pl.ds(ib, _CPW)], hh)
        pltpu.sync_copy(rh_hbm.at[pl.ds(ib, _CPW)], rh)
        pltpu.sync_copy(th_hbm.at[pl.ds(ib, _CPW)], th)
        rb0 = wid * _RPW
        pltpu.sync_copy(hp_hbm.at[pl.ds(rb0, _RPW)], hp)
        pltpu.sync_copy(rp_hbm.at[pl.ds(rb0, _RPW)], rp)
        pltpu.sync_copy(tp_hbm.at[pl.ds(rb0, _RPW)], tp)
        hb0 = wid * _HPW
        pltpu.sync_copy(pp_hbm.at[pl.ds(hb0, _HPW)], ppv)
        pltpu.sync_copy(ap_hbm.at[pl.ds(hb0, _HPW)], apv)
        pltpu.sync_copy(prm_hbm, prm_v)

        lanes = lax.iota(jnp.int32, _L)

        def fire(kk, bh, br, bt, sem):
            pltpu.async_copy(ent_hbm.at[hh.at[kk]], bh, sem)
            pltpu.async_copy(rel_hbm.at[rh.at[kk]], br, sem)
            pltpu.async_copy(ent_hbm.at[th.at[kk]], bt, sem)

        def drain(sem, buf):
            for _ in range(3):
                pltpu.make_async_copy(ent_hbm.at[pl.ds(0, _CH)], buf, sem).wait()

        def compute(kk, bh, br, bt):
            def grp(g, carry):
                off = kk * _CH + g * _L
                rows = lanes + g * _L
                dh = hp[pl.ds(off, _L)]
                dr = rp[pl.ds(off, _L)]
                dt = tp[pl.ds(off, _L)]
                acc = jnp.zeros((_L,), jnp.float32)
                for dd in range(_D):
                    dsp = jnp.full((_L,), dd, jnp.int32)
                    hv = plsc.load_gather(bh, [rows, dh + dsp])
                    rv = plsc.load_gather(br, [rows, dr + dsp])
                    tv = plsc.load_gather(bt, [rows, dt + dsp])
                    e = hv + rv - tv
                    acc = acc + e * e
                ss[pl.ds(off, _L)] = acc
                return carry

            lax.fori_loop(0, _CH // _L, grp, 0)

        fire(0, ha, ra, ta, semA)

        def chunk_pair(c, carry):
            k0 = c * 2
            fire(k0 + 1, hb, rb, tb, semB)
            drain(semA, ha)
            compute(k0, ha, ra, ta)

            @pl.when(c < (_CPW // 2 - 1))
            def _():
                fire(k0 + 2, ha, ra, ta, semA)

            drain(semB, hb)
            compute(k0 + 1, hb, rb, tb)
            return carry

        lax.fori_loop(0, _CPW // 2, chunk_pair, 0)

        alpha_s = _splat(prm_v, 0)
        beta1_s = _splat(prm_v, 1)
        l1_s = _splat(prm_v, 2)
        l2_s = _splat(prm_v, 3)
        l3_s = _splat(prm_v, 4)

        def fin(g, acc):
            sp = ss[pl.ds(g * _L, _L)]
            sn = ss[pl.ds(_HPW + g * _L, _L)]
            d = _vsqrt(sp + 1e-12) - _vsqrt(sn + 1e-12) + 1.0
            lt = jnp.where(d < 0.0, beta1_s, alpha_s)
            ppx = ppv[pl.ds(g * _L, _L)]
            apx = apv[pl.ds(g * _L, _L)]
            sg = 1.0 / (1.0 + jnp.exp(-apx))
            cw = l1_s * lt + l2_s * ppx + l3_s * sg
            return acc + jnp.maximum(d * cw, 0.0)

        acc = lax.fori_loop(0, _HPW // _L, fin, jnp.zeros((_L,), jnp.float32))
        accv[pl.ds(0, _L)] = acc
        pltpu.sync_copy(accv, out_hbm.at[pl.ds(wid * _L, _L)])
        pltpu.sync_copy(ss, ss_hbm.at[pl.ds(wid * _RPW, _RPW)])

    return k(entv, relv, hh2, rh2, th2, hp1, rp1, tp1, pp, ap, prm)


def _tc_sum(partials):
    def body(x_ref, o_ref):
        o_ref[0, 0] = jnp.sum(x_ref[...]) * (1.0 / _B)

    out = pl.pallas_call(
        body,
        out_shape=jax.ShapeDtypeStruct((1, 1), jnp.float32),
        in_specs=[pl.BlockSpec(memory_space=pltpu.VMEM)],
        out_specs=pl.BlockSpec(memory_space=pltpu.SMEM),
    )(partials)
    return out[0, 0]


def kernel(posX, negX, entityEmbedding, relationEmbedding, PP, AP,
           alpha, beta, sigma, lambda1, lambda2, lambda3):
    def interleave(a, b):
        return jnp.concatenate(
            [a.reshape(_NW, _HPW), b.reshape(_NW, _HPW)], axis=1).reshape(-1)

    h_all = interleave(posX[:, 0], negX[:, 0])
    r_all = interleave(posX[:, 1], negX[:, 1])
    t_all = interleave(posX[:, 2], negX[:, 2])

    hh2 = (h_all >> 1).reshape(_TOT // _CH, _CH)
    rh2 = (r_all >> 1).reshape(_TOT // _CH, _CH)
    th2 = (t_all >> 1).reshape(_TOT // _CH, _CH)
    hp1 = (h_all & 1) * _D
    rp1 = (r_all & 1) * _D
    tp1 = (t_all & 1) * _D

    entv = entityEmbedding.reshape(-1, 2 * _D)
    relv = relationEmbedding.reshape(-1, 2 * _D)

    prm = jnp.concatenate([
        jnp.stack([alpha, 1.0 + beta, lambda1, lambda2, lambda3])
        .astype(jnp.float32),
        jnp.zeros((11,), jnp.float32),
    ])

    partials, ss = _sc_loss(entv, relv, hh2, rh2, th2, hp1, rp1, tp1,
                            PP, AP, prm)
    del partials
    ssr = ss.reshape(_NW, 2, _HPW)
    ssp = ssr[:, 0, :].reshape(128, 128)
    ssn = ssr[:, 1, :].reshape(128, 128)

    def body(par_ref, sp_ref, sn_ref, pp_ref, ap_ref, o_ref):
        a = par_ref[0]
        b1 = par_ref[1]
        l1 = par_ref[2]
        l2 = par_ref[3]
        l3 = par_ref[4]
        pos = jnp.sqrt(sp_ref[...] + 1e-12)
        neg = jnp.sqrt(sn_ref[...] + 1e-12)
        d = pos - neg + 1.0
        lt = jnp.where(d < 0, b1, a)
        cw = l1 * lt + l2 * pp_ref[...] + l3 * (1.0 / (1.0 + jnp.exp(-ap_ref[...])))
        o_ref[0, 0] = jnp.sum(jnp.maximum(d * cw, 0.0)) * (1.0 / _B)

    out = pl.pallas_call(
        body,
        out_shape=jax.ShapeDtypeStruct((1, 1), jnp.float32),
        in_specs=[pl.BlockSpec(memory_space=pltpu.SMEM)] +
                 [pl.BlockSpec(memory_space=pltpu.VMEM)] * 4,
        out_specs=pl.BlockSpec(memory_space=pltpu.SMEM),
    )(prm[:5], ssp, ssn,
      PP.reshape(_NW, _HPW).reshape(128, 128),
      AP.reshape(_NW, _HPW).reshape(128, 128))
    return out[0, 0]


# speedup vs baseline: 1.5390x; 1.5338x over previous
import functools

import jax
import jax.numpy as jnp
from jax import lax
from jax.experimental import pallas as pl
from jax.experimental.pallas import tpu as pltpu
from jax.experimental.pallas import tpu_sc as plsc

_B = 16384
_D = 64
_TOT = 2 * _B
_CH = 128
_NW = 32
_RPW = _TOT // _NW
_HPW = _RPW // 2
_CPW = _RPW // _CH
_L = 16
_GPC = _CH // _L


def _sc_sumsq(ent, rel, h1, r1, t1):
    mesh = plsc.VectorSubcoreMesh(core_axis_name="c", subcore_axis_name="s")

    @functools.partial(
        pl.kernel,
        mesh=mesh,
        out_type=jax.ShapeDtypeStruct((_TOT,), jnp.float32),
        compiler_params=pltpu.CompilerParams(needs_layout_passes=False),
        scratch_types=[
            pltpu.VMEM((_RPW,), jnp.int32),
            pltpu.VMEM((_RPW,), jnp.int32),
            pltpu.VMEM((_RPW,), jnp.int32),
            pltpu.VMEM((_CH, 2 * _D), jnp.float32),
            pltpu.VMEM((_CH, 2 * _D), jnp.float32),
            pltpu.VMEM((_CH, 2 * _D), jnp.float32),
            pltpu.VMEM((_CH, 2 * _D), jnp.float32),
            pltpu.VMEM((_CH, 2 * _D), jnp.float32),
            pltpu.VMEM((_CH, 2 * _D), jnp.float32),
            pltpu.VMEM((64 * 128,), jnp.int32),
            pltpu.VMEM((_RPW,), jnp.float32),
            pltpu.SemaphoreType.DMA,
            pltpu.SemaphoreType.DMA,
        ],
    )
    def k(ent_hbm, rel_hbm, h_hbm, r_hbm, t_hbm, out_hbm,
          hi, ri, ti, ha, ra, ta, hb, rb, tb, dmy, ss, semA, semB):
        wid = lax.axis_index("s") * 2 + lax.axis_index("c")
        rb0 = wid * _RPW
        pltpu.sync_copy(h_hbm.at[pl.ds(rb0, _RPW)], hi)
        pltpu.sync_copy(r_hbm.at[pl.ds(rb0, _RPW)], ri)
        pltpu.sync_copy(t_hbm.at[pl.ds(rb0, _RPW)], ti)

        lanes = lax.iota(jnp.int32, _L)

        def fire(kk, bh, br, bt, sem):
            def grp(g, carry):
                off = kk * _CH + g * _L
                hv = hi[pl.ds(off, _L)]
                rv = ri[pl.ds(off, _L)]
                tv = ti[pl.ds(off, _L)]
                for j in range(_L):
                    row = g * _L + j
                    pltpu.async_copy(
                        ent_hbm.at[hv[j]], bh.at[row, pl.ds(0, _D)], sem)
                    pltpu.async_copy(
                        rel_hbm.at[rv[j]], br.at[row, pl.ds(0, _D)], sem)
                    pltpu.async_copy(
                        ent_hbm.at[tv[j]], bt.at[row, pl.ds(0, _D)], sem)
                return carry

            lax.fori_loop(0, _GPC, grp, 0)

        def drain(sem):
            for _ in range(3):
                pltpu.make_async_copy(h_hbm.at[pl.ds(0, 64 * 128)],
                                      dmy, sem).wait()

        def compute(kk, bh, br, bt):
            def grp(g, carry):
                off = kk * _CH + g * _L
                rows = lanes + g * _L
                acc = jnp.zeros((_L,), jnp.float32)
                for dd in range(_D):
                    dsp = jnp.full((_L,), dd, jnp.int32)
                    hv = plsc.load_gather(bh, [rows, dsp])
                    rv = plsc.load_gather(br, [rows, dsp])
                    tv = plsc.load_gather(bt, [rows, dsp])
                    e = hv + rv - tv
                    acc = acc + e * e
                ss[pl.ds(off, _L)] = acc
                return carry

            lax.fori_loop(0, _GPC, grp, 0)

        fire(0, ha, ra, ta, semA)

        def chunk_pair(c, carry):
            k0 = c * 2
            fire(k0 + 1, hb, rb, tb, semB)
            drain(semA)
            compute(k0, ha, ra, ta)

            @pl.when(c < (_CPW // 2 - 1))
            def _():
                fire(k0 + 2, ha, ra, ta, semA)

            drain(semB)
            compute(k0 + 1, hb, rb, tb)
            return carry

        lax.fori_loop(0, _CPW // 2, chunk_pair, 0)
        pltpu.sync_copy(ss, out_hbm.at[pl.ds(wid * _RPW, _RPW)])

    return k(ent, rel, h1, r1, t1)


def kernel(posX, negX, entityEmbedding, relationEmbedding, PP, AP,
           alpha, beta, sigma, lambda1, lambda2, lambda3):
    def interleave(a, b):
        return jnp.concatenate(
            [a.reshape(_NW, _HPW), b.reshape(_NW, _HPW)], axis=1).reshape(-1)

    h1 = interleave(posX[:, 0], negX[:, 0])
    r1 = interleave(posX[:, 1], negX[:, 1])
    t1 = interleave(posX[:, 2], negX[:, 2])

    ss = _sc_sumsq(entityEmbedding, relationEmbedding, h1, r1, t1)
    ssr = ss.reshape(_NW, 2, _HPW)
    ssp = ssr[:, 0, :].reshape(128, 128)
    ssn = ssr[:, 1, :].reshape(128, 128)

    prm = jnp.stack([alpha, 1.0 + beta, lambda1, lambda2, lambda3]).astype(
        jnp.float32)

    def body(par_ref, sp_ref, sn_ref, pp_ref, ap_ref, o_ref):
        a = par_ref[0]
        b1 = par_ref[1]
        l1 = par_ref[2]
        l2 = par_ref[3]
        l3 = par_ref[4]
        pos = jnp.sqrt(sp_ref[...] + 1e-12)
        neg = jnp.sqrt(sn_ref[...] + 1e-12)
        d = pos - neg + 1.0
        lt = jnp.where(d < 0, b1, a)
        cw = l1 * lt + l2 * pp_ref[...] + l3 * (1.0 / (1.0 + jnp.exp(-ap_ref[...])))
        o_ref[0, 0] = jnp.sum(jnp.maximum(d * cw, 0.0)) * (1.0 / _B)

    out = pl.pallas_call(
        body,
        out_shape=jax.ShapeDtypeStruct((1, 1), jnp.float32),
        in_specs=[pl.BlockSpec(memory_space=pltpu.SMEM)] +
                 [pl.BlockSpec(memory_space=pltpu.VMEM)] * 4,
        out_specs=pl.BlockSpec(memory_space=pltpu.SMEM),
    )(prm, ssp, ssn, PP.reshape(128, 128), AP.reshape(128, 128))
    return out[0, 0]
